# baseline (speedup 1.0000x reference)
import functools

import jax
import jax.numpy as jnp
from jax import lax
from jax.experimental import pallas as pl
from jax.experimental.pallas import tpu as pltpu
from jax.experimental.pallas import tpu_sc as plsc

ALPHA = 0.3
_BS = 1024
_L = 16
_NT = 16


def _copy_body(hid_ref, out_ref):
    out_ref[...] = hid_ref[...]


def _tc_copy(flat):
    rows, d = flat.shape
    return pl.pallas_call(
        _copy_body,
        grid=(rows // _BS,),
        in_specs=[pl.BlockSpec((_BS, d), lambda i: (i, 0))],
        out_specs=pl.BlockSpec((_BS, d), lambda i: (i, 0)),
        out_shape=jax.ShapeDtypeStruct((rows, d), flat.dtype),
    )(flat)


def _sc_body(hid_ref, attr_ref, out_ref, h_v, attr_v, row_v, stage_v, shared_v,
             gather_idx_v, win_v, sem):
    rows, d = hid_ref.shape
    k = attr_ref.shape[0]
    nchunk = d // _L
    per_tile = k // _NT
    half = per_tile // 2

    wid = lax.axis_index("s")

    pltpu.sync_copy(hid_ref.at[rows // 2 - 1], h_v.at[0])
    pltpu.sync_copy(hid_ref.at[rows - 1], h_v.at[1])

    def score_half(hf, carry):
        base = wid * per_tile + hf * half
        pltpu.sync_copy(attr_ref.at[pl.ds(base, half)], attr_v)

        def one_attr(a, c):
            bs0, bi0, bs1, bi1 = c

            def dot_chunks(cc, accs):
                a0, a1 = accs
                av = attr_v[a, pl.ds(cc * _L, _L)]
                a0 = a0 + av * h_v[0, pl.ds(cc * _L, _L)]
                a1 = a1 + av * h_v[1, pl.ds(cc * _L, _L)]
                return a0, a1

            z = jnp.zeros((_L,), jnp.float32)
            acc0, acc1 = lax.fori_loop(0, nchunk, dot_chunks, (z, z))
            s0 = plsc.cumsum(acc0)[_L - 1]
            s1 = plsc.cumsum(acc1)[_L - 1]
            gidx = base + a
            better0 = s0 > bs0
            better1 = s1 > bs1
            bs0 = jnp.where(better0, s0, bs0)
            bi0 = jnp.where(better0, gidx, bi0)
            bs1 = jnp.where(better1, s1, bs1)
            bi1 = jnp.where(better1, gidx, bi1)
            return bs0, bi0, bs1, bi1

        return lax.fori_loop(0, half, one_attr, carry)

    neg = jnp.float32(-3e38)
    best = (neg, jnp.int32(0), neg, jnp.int32(0))
    best = score_half(0, best)
    bs0, bi0, bs1, bi1 = score_half(1, best)

    lane = lax.iota(jnp.int32, _L)
    pub = jnp.where(lane == 0, bs0,
                    jnp.where(lane == 1, bs1,
                              jnp.where(lane == 2, bi0.astype(jnp.float32),
                                        jnp.where(lane == 3,
                                                  bi1.astype(jnp.float32),
                                                  0.0))))
    stage_v[...] = pub
    pltpu.sync_copy(stage_v, shared_v.at[wid])
    plsc.subcore_barrier()
    pltpu.sync_copy(shared_v, win_v)

    gbs0, gbi0 = neg, jnp.int32(0)
    gbs1, gbi1 = neg, jnp.int32(0)
    for t in range(_NT):
        rv = win_v[t, :]
        v0 = rv[0]
        v1 = rv[1]
        i0 = rv[2].astype(jnp.int32)
        i1 = rv[3].astype(jnp.int32)
        b0 = v0 > gbs0
        b1 = v1 > gbs1
        gbs0 = jnp.where(b0, v0, gbs0)
        gbi0 = jnp.where(b0, i0, gbi0)
        gbs1 = jnp.where(b1, v1, gbs1)
        gbi1 = jnp.where(b1, i1, gbi1)

    def finalize(b, gidx):
        @pl.when(wid == gidx // per_tile)
        def _():
            gather_idx_v[...] = jnp.full((_L,), gidx, jnp.int32)
            pltpu.async_copy(attr_ref.at[gather_idx_v], row_v, sem).wait()
            pltpu.sync_copy(row_v.at[0], out_ref.at[b])

    finalize(0, gbi0)
    finalize(1, gbi1)


def _sc_rows(flat, attractors):
    d = flat.shape[1]
    k = attractors.shape[0]
    half = k // _NT // 2
    mesh = plsc.VectorSubcoreMesh(
        core_axis_name="c", subcore_axis_name="s", num_cores=1)
    f = pl.kernel(
        _sc_body,
        out_type=jax.ShapeDtypeStruct((2, d), jnp.float32),
        mesh=mesh,
        compiler_params=pltpu.CompilerParams(needs_layout_passes=False),
        scratch_types=[
            pltpu.VMEM((2, d), jnp.float32),
            pltpu.VMEM((half, d), jnp.float32),
            pltpu.VMEM((_L, d), jnp.float32),
            pltpu.VMEM((_L,), jnp.float32),
            pltpu.VMEM_SHARED((_NT, _L), jnp.float32),
            pltpu.VMEM((_L,), jnp.int32),
            pltpu.VMEM((_NT, _L), jnp.float32),
            pltpu.SemaphoreType.DMA,
        ],
    )
    return f(flat, attractors)


def _merge_body(cp_ref, rows_ref, out_ref):
    i = pl.program_id(0)
    out_ref[...] = cp_ref[...]
    h = cp_ref[7, :]
    norm = jnp.sqrt(jnp.sum(h * h))
    out_ref[7, :] = (1.0 - ALPHA) * h + (ALPHA * norm) * rows_ref[i, :]


def _merge(copied, new_rows):
    rows, d = copied.shape
    nblk = rows // 8
    return pl.pallas_call(
        _merge_body,
        grid=(2,),
        in_specs=[
            pl.BlockSpec((8, d), lambda i: ((i + 1) * (nblk // 2) - 1, 0)),
            pl.BlockSpec((2, d), lambda i: (0, 0)),
        ],
        out_specs=pl.BlockSpec((8, d), lambda i: ((i + 1) * (nblk // 2) - 1, 0)),
        out_shape=jax.ShapeDtypeStruct((rows, d), copied.dtype),
        input_output_aliases={0: 0},
    )(copied, new_rows)


def kernel(hidden_states, attractors):
    b, s, d = hidden_states.shape
    flat = hidden_states.reshape(b * s, d)
    copied = _tc_copy(flat)
    new_rows = _sc_rows(flat, attractors)
    out = _merge(copied, new_rows)
    return out.reshape(b, s, d)
